# baseline (speedup 1.0000x reference)
import functools

import jax
import jax.numpy as jnp
from jax import lax
from jax.experimental import pallas as pl
from jax.experimental.pallas import tpu as pltpu, tpu_sc as plsc


_BLK = 1024


def _three_nn_kernel(x1_ref, x2t_ref, idx_ref, m_ref):
    n2 = x2t_ref.shape[2]
    x1 = x1_ref[0]
    x2 = x2t_ref[0]
    dx = x1[:, 0:1] - x2[0:1, :]
    dy = x1[:, 1:2] - x2[1:2, :]
    dz = x1[:, 2:3] - x2[2:3, :]
    d = dx * dx + dy * dy + dz * dz

    inf = jnp.float32(jnp.inf)
    m1 = jnp.min(d, axis=1, keepdims=True)
    m2 = jnp.min(jnp.where(d > m1, d, inf), axis=1, keepdims=True)
    m3 = jnp.min(jnp.where(d > m2, d, inf), axis=1, keepdims=True)

    iota = lax.broadcasted_iota(jnp.int32, d.shape, 1)
    off = pl.program_id(0) * n2
    i1 = jnp.min(jnp.where(d == m1, iota, n2), axis=1, keepdims=True)
    i2 = jnp.min(jnp.where(d == m2, iota, n2), axis=1, keepdims=True)
    i3 = jnp.min(jnp.where(d == m3, iota, n2), axis=1, keepdims=True)
    idx_ref[0] = jnp.concatenate([i1, i2, i3], axis=1) + off
    m_ref[0] = jnp.concatenate([m1, m2, m3], axis=1)


def _three_nn(xyz1, xyz2t):
    B, N1, _ = xyz1.shape
    N2 = xyz2t.shape[2]
    return pl.pallas_call(
        _three_nn_kernel,
        grid=(B, N1 // _BLK),
        in_specs=[
            pl.BlockSpec((1, _BLK, 3), lambda b, j: (b, j, 0)),
            pl.BlockSpec((1, 3, N2), lambda b, j: (b, 0, 0)),
        ],
        out_specs=[
            pl.BlockSpec((1, _BLK, 3), lambda b, j: (b, j, 0)),
            pl.BlockSpec((1, _BLK, 3), lambda b, j: (b, j, 0)),
        ],
        out_shape=[
            jax.ShapeDtypeStruct((B, N1, 3), jnp.int32),
            jax.ShapeDtypeStruct((B, N1, 3), jnp.float32),
        ],
    )(xyz1, xyz2t)


def _sc_gather(table, idx_flat):
    V, D = table.shape
    Q3 = idx_flat.shape[0]
    NC, NS = 2, 16
    NW = NC * NS
    per_w = Q3 // NW
    chunk = 512
    mesh = plsc.VectorSubcoreMesh(core_axis_name="c", subcore_axis_name="s",
                                  num_cores=NC, num_subcores=NS)

    @functools.partial(
        pl.kernel, mesh=mesh,
        out_type=jax.ShapeDtypeStruct((Q3, D), jnp.float32),
        scratch_types=[
            pltpu.VMEM((chunk,), jnp.int32),
            pltpu.VMEM((chunk, D), jnp.float32),
            pltpu.SemaphoreType.DMA,
        ],
    )
    def k(table_hbm, idx_hbm, out_hbm, idx_v, rows_v, sem):
        wid = lax.axis_index("s") * NC + lax.axis_index("c")
        for i in range(per_w // chunk):
            base = wid * per_w + i * chunk
            pltpu.sync_copy(idx_hbm.at[pl.ds(base, chunk)], idx_v)
            pltpu.async_copy(table_hbm.at[idx_v], rows_v, sem).wait()
            pltpu.sync_copy(rows_v, out_hbm.at[pl.ds(base, chunk)])

    return k(table, idx_flat)


def _blend_mlp_kernel(g_ref, m_ref, points1_ref,
                      w0a_ref, w0b_ref, b0_ref, w1_ref, b1_ref, out_ref):
    g = g_ref[0]
    m = m_ref[0]
    c2 = w0a_ref.shape[0]
    inv1 = 1.0 / jnp.maximum(m[:, 0:1], 1e-10)
    inv2 = 1.0 / jnp.maximum(m[:, 1:2], 1e-10)
    inv3 = 1.0 / jnp.maximum(m[:, 2:3], 1e-10)
    s = 1.0 / (inv1 + inv2 + inv3)
    interp = (g[:, 0:c2] * (inv1 * s) + g[:, 128:128 + c2] * (inv2 * s)
              + g[:, 256:256 + c2] * (inv3 * s))
    h = jnp.maximum(
        jnp.dot(interp, w0a_ref[...], preferred_element_type=jnp.float32)
        + jnp.dot(points1_ref[0], w0b_ref[...], preferred_element_type=jnp.float32)
        + b0_ref[...], 0.0)
    out_ref[0] = jnp.maximum(
        jnp.dot(h, w1_ref[...], preferred_element_type=jnp.float32)
        + b1_ref[...], 0.0)


@jax.jit
def kernel(xyz1, points1, xyz2, points2, W0, b0, W1, b1):
    B, N1, _ = xyz1.shape
    _, N2, C2 = points2.shape
    C1 = points1.shape[2]
    xyz2t = jnp.swapaxes(xyz2, 1, 2)

    idx, m = _three_nn(xyz1, xyz2t)

    table = jnp.pad(points2.reshape(B * N2, C2), ((0, 0), (0, 128 - C2)))
    gathered = _sc_gather(table, idx.reshape(B * N1 * 3))
    g = gathered.reshape(B, N1, 3 * 128)

    w0a, w0b = W0[:C2], W0[C2:]
    b0r = b0.reshape(1, -1)
    b1r = b1.reshape(1, -1)
    return pl.pallas_call(
        _blend_mlp_kernel,
        grid=(B, N1 // _BLK),
        in_specs=[
            pl.BlockSpec((1, _BLK, 3 * 128), lambda b, j: (b, j, 0)),
            pl.BlockSpec((1, _BLK, 3), lambda b, j: (b, j, 0)),
            pl.BlockSpec((1, _BLK, C1), lambda b, j: (b, j, 0)),
            pl.BlockSpec((C2, W0.shape[1]), lambda b, j: (0, 0)),
            pl.BlockSpec((C1, W0.shape[1]), lambda b, j: (0, 0)),
            pl.BlockSpec((1, W0.shape[1]), lambda b, j: (0, 0)),
            pl.BlockSpec(W1.shape, lambda b, j: (0, 0)),
            pl.BlockSpec((1, W1.shape[1]), lambda b, j: (0, 0)),
        ],
        out_specs=pl.BlockSpec((1, _BLK, W1.shape[1]), lambda b, j: (b, j, 0)),
        out_shape=jax.ShapeDtypeStruct((B, N1, W1.shape[1]), jnp.float32),
    )(g, m, points1, w0a, w0b, b0r, W1, b1r)


# speedup vs baseline: 1.8750x; 1.8750x over previous
import jax
import jax.numpy as jnp
from jax.experimental import pallas as pl


_BLK = 512


def _fp_kernel(x1aug_ref, x2aug_ref, points1_ref, points2_ref,
               w0a_ref, w0b_ref, b0_ref, w1_ref, b1_ref, out_ref):
    x1 = x1aug_ref[0]
    x2 = x2aug_ref[0]
    dx = x1[:, 0:1] - x2[0:1, :]
    dy = x1[:, 1:2] - x2[1:2, :]
    dz = x1[:, 2:3] - x2[2:3, :]
    d = dx * dx + dy * dy + dz * dz

    inf = jnp.float32(jnp.inf)
    m1 = jnp.min(d, axis=1, keepdims=True)
    m2 = jnp.min(jnp.where(d > m1, d, inf), axis=1, keepdims=True)
    m3 = jnp.min(jnp.where(d > m2, d, inf), axis=1, keepdims=True)

    inv_sum = (1.0 / jnp.maximum(m1, 1e-10) + 1.0 / jnp.maximum(m2, 1e-10)
               + 1.0 / jnp.maximum(m3, 1e-10))
    sel = jnp.where(d <= m3, 1.0 / jnp.maximum(d, 1e-10), 0.0)
    interp = jnp.dot(sel, points2_ref[0],
                     preferred_element_type=jnp.float32) * (1.0 / inv_sum)

    h = jnp.maximum(
        jnp.dot(interp, w0a_ref[...], preferred_element_type=jnp.float32)
        + jnp.dot(points1_ref[0], w0b_ref[...], preferred_element_type=jnp.float32)
        + b0_ref[...], 0.0)
    out_ref[0] = jnp.maximum(
        jnp.dot(h, w1_ref[...], preferred_element_type=jnp.float32)
        + b1_ref[...], 0.0)


@jax.jit
def kernel(xyz1, points1, xyz2, points2, W0, b0, W1, b1):
    B, N1, _ = xyz1.shape
    _, N2, C2 = points2.shape
    C1 = points1.shape[2]
    xyz2t = jnp.swapaxes(xyz2, 1, 2)
    w0a, w0b = W0[:C2], W0[C2:]
    b0r = b0.reshape(1, -1)
    b1r = b1.reshape(1, -1)
    grid = (B, N1 // _BLK)
    return pl.pallas_call(
        _fp_kernel,
        grid=grid,
        in_specs=[
            pl.BlockSpec((1, _BLK, 3), lambda b, j: (b, j, 0)),
            pl.BlockSpec((1, 3, N2), lambda b, j: (b, 0, 0)),
            pl.BlockSpec((1, _BLK, C1), lambda b, j: (b, j, 0)),
            pl.BlockSpec((1, N2, C2), lambda b, j: (b, 0, 0)),
            pl.BlockSpec((C2, W0.shape[1]), lambda b, j: (0, 0)),
            pl.BlockSpec((C1, W0.shape[1]), lambda b, j: (0, 0)),
            pl.BlockSpec((1, W0.shape[1]), lambda b, j: (0, 0)),
            pl.BlockSpec(W1.shape, lambda b, j: (0, 0)),
            pl.BlockSpec((1, W1.shape[1]), lambda b, j: (0, 0)),
        ],
        out_specs=pl.BlockSpec((1, _BLK, W1.shape[1]), lambda b, j: (b, j, 0)),
        out_shape=jax.ShapeDtypeStruct((B, N1, W1.shape[1]), jnp.float32),
    )(xyz1, xyz2t, points1, points2, w0a, w0b, b0r, W1, b1r)


# speedup vs baseline: 2.0209x; 1.0778x over previous
import jax
import jax.numpy as jnp
from jax.experimental import pallas as pl


_BLK = 2048


def _fp_kernel(x1aug_ref, x2aug_ref, points1_ref, points2_ref,
               w0a_ref, w0b_ref, b0_ref, w1_ref, b1_ref, out_ref):
    x1 = x1aug_ref[0]
    x2 = x2aug_ref[0]
    dx = x1[:, 0:1] - x2[0:1, :]
    dy = x1[:, 1:2] - x2[1:2, :]
    dz = x1[:, 2:3] - x2[2:3, :]
    d = dx * dx + dy * dy + dz * dz

    inf = jnp.float32(jnp.inf)
    m1 = jnp.min(d, axis=1, keepdims=True)
    m2 = jnp.min(jnp.where(d > m1, d, inf), axis=1, keepdims=True)
    m3 = jnp.min(jnp.where(d > m2, d, inf), axis=1, keepdims=True)

    inv_sum = (1.0 / jnp.maximum(m1, 1e-10) + 1.0 / jnp.maximum(m2, 1e-10)
               + 1.0 / jnp.maximum(m3, 1e-10))
    sel = jnp.where(d <= m3, 1.0 / jnp.maximum(d, 1e-10), 0.0)
    interp = jnp.dot(sel, points2_ref[0],
                     preferred_element_type=jnp.float32) * (1.0 / inv_sum)

    h = jnp.maximum(
        jnp.dot(interp, w0a_ref[...], preferred_element_type=jnp.float32)
        + jnp.dot(points1_ref[0], w0b_ref[...], preferred_element_type=jnp.float32)
        + b0_ref[...], 0.0)
    out_ref[0] = jnp.maximum(
        jnp.dot(h, w1_ref[...], preferred_element_type=jnp.float32)
        + b1_ref[...], 0.0)


@jax.jit
def kernel(xyz1, points1, xyz2, points2, W0, b0, W1, b1):
    B, N1, _ = xyz1.shape
    _, N2, C2 = points2.shape
    C1 = points1.shape[2]
    xyz2t = jnp.swapaxes(xyz2, 1, 2)
    w0a, w0b = W0[:C2], W0[C2:]
    b0r = b0.reshape(1, -1)
    b1r = b1.reshape(1, -1)
    grid = (B, N1 // _BLK)
    return pl.pallas_call(
        _fp_kernel,
        grid=grid,
        in_specs=[
            pl.BlockSpec((1, _BLK, 3), lambda b, j: (b, j, 0)),
            pl.BlockSpec((1, 3, N2), lambda b, j: (b, 0, 0)),
            pl.BlockSpec((1, _BLK, C1), lambda b, j: (b, j, 0)),
            pl.BlockSpec((1, N2, C2), lambda b, j: (b, 0, 0)),
            pl.BlockSpec((C2, W0.shape[1]), lambda b, j: (0, 0)),
            pl.BlockSpec((C1, W0.shape[1]), lambda b, j: (0, 0)),
            pl.BlockSpec((1, W0.shape[1]), lambda b, j: (0, 0)),
            pl.BlockSpec(W1.shape, lambda b, j: (0, 0)),
            pl.BlockSpec((1, W1.shape[1]), lambda b, j: (0, 0)),
        ],
        out_specs=pl.BlockSpec((1, _BLK, W1.shape[1]), lambda b, j: (b, j, 0)),
        out_shape=jax.ShapeDtypeStruct((B, N1, W1.shape[1]), jnp.float32),
    )(xyz1, xyz2t, points1, points2, w0a, w0b, b0r, W1, b1r)


# speedup vs baseline: 2.0745x; 1.0265x over previous
import jax
import jax.numpy as jnp
from jax.experimental import pallas as pl


_BLK = 4096


def _fp_kernel(x1aug_ref, x2aug_ref, points1_ref, points2_ref,
               w0a_ref, w0b_ref, b0_ref, w1_ref, b1_ref, out_ref):
    x1 = x1aug_ref[0]
    x2 = x2aug_ref[0]
    dx = x1[:, 0:1] - x2[0:1, :]
    dy = x1[:, 1:2] - x2[1:2, :]
    dz = x1[:, 2:3] - x2[2:3, :]
    d = dx * dx + dy * dy + dz * dz

    inf = jnp.float32(jnp.inf)
    m1 = jnp.min(d, axis=1, keepdims=True)
    m2 = jnp.min(jnp.where(d > m1, d, inf), axis=1, keepdims=True)
    m3 = jnp.min(jnp.where(d > m2, d, inf), axis=1, keepdims=True)

    inv_sum = (1.0 / jnp.maximum(m1, 1e-10) + 1.0 / jnp.maximum(m2, 1e-10)
               + 1.0 / jnp.maximum(m3, 1e-10))
    sel = jnp.where(d <= m3, 1.0 / jnp.maximum(d, 1e-10), 0.0)
    interp = jnp.dot(sel, points2_ref[0],
                     preferred_element_type=jnp.float32) * (1.0 / inv_sum)

    h = jnp.maximum(
        jnp.dot(interp, w0a_ref[...], preferred_element_type=jnp.float32)
        + jnp.dot(points1_ref[0], w0b_ref[...], preferred_element_type=jnp.float32)
        + b0_ref[...], 0.0)
    out_ref[0] = jnp.maximum(
        jnp.dot(h, w1_ref[...], preferred_element_type=jnp.float32)
        + b1_ref[...], 0.0)


@jax.jit
def kernel(xyz1, points1, xyz2, points2, W0, b0, W1, b1):
    B, N1, _ = xyz1.shape
    _, N2, C2 = points2.shape
    C1 = points1.shape[2]
    xyz2t = jnp.swapaxes(xyz2, 1, 2)
    w0a, w0b = W0[:C2], W0[C2:]
    b0r = b0.reshape(1, -1)
    b1r = b1.reshape(1, -1)
    grid = (B, N1 // _BLK)
    return pl.pallas_call(
        _fp_kernel,
        grid=grid,
        in_specs=[
            pl.BlockSpec((1, _BLK, 3), lambda b, j: (b, j, 0)),
            pl.BlockSpec((1, 3, N2), lambda b, j: (b, 0, 0)),
            pl.BlockSpec((1, _BLK, C1), lambda b, j: (b, j, 0)),
            pl.BlockSpec((1, N2, C2), lambda b, j: (b, 0, 0)),
            pl.BlockSpec((C2, W0.shape[1]), lambda b, j: (0, 0)),
            pl.BlockSpec((C1, W0.shape[1]), lambda b, j: (0, 0)),
            pl.BlockSpec((1, W0.shape[1]), lambda b, j: (0, 0)),
            pl.BlockSpec(W1.shape, lambda b, j: (0, 0)),
            pl.BlockSpec((1, W1.shape[1]), lambda b, j: (0, 0)),
        ],
        out_specs=pl.BlockSpec((1, _BLK, W1.shape[1]), lambda b, j: (b, j, 0)),
        out_shape=jax.ShapeDtypeStruct((B, N1, W1.shape[1]), jnp.float32),
    )(xyz1, xyz2t, points1, points2, w0a, w0b, b0r, W1, b1r)
